# baseline (speedup 1.0000x reference)
import functools

import jax
import jax.numpy as jnp
from jax import lax
from jax.experimental import pallas as pl
from jax.experimental.pallas import tpu as pltpu
from jax.experimental.pallas import tpu_sc as plsc

_PAD = 1
_EPS = 1e-12
_L = 16


def _rsqrt_scalar(x):
    i = lax.bitcast_convert_type(x, jnp.int32)
    y = lax.bitcast_convert_type(jnp.int32(0x5F3759DF) - (i >> 1), jnp.float32)
    for _ in range(4):
        y = y * (1.5 - 0.5 * x * y * y)
    return y


def kernel(input_ids, word_emb, token_type_emb, position_emb, ln_gamma, ln_beta):
    B, S = input_ids.shape
    V, H = word_emb.shape
    NH = H // _L

    mesh = plsc.VectorSubcoreMesh(core_axis_name="c", subcore_axis_name="s")
    NC = mesh.num_cores
    NW = NC * mesh.num_subcores
    assert B == NW, (B, NW)

    CH = 16
    NCHUNK = S // CH
    T = 8

    @functools.partial(
        pl.kernel,
        out_type=jax.ShapeDtypeStruct((B, S, H), jnp.float32),
        mesh=mesh,
        compiler_params=pltpu.CompilerParams(needs_layout_passes=False),
        scratch_types=[
            pltpu.VMEM((S,), jnp.int32),
            pltpu.VMEM((S,), jnp.int32),
            pltpu.VMEM((4, CH, H), jnp.float32),
            pltpu.VMEM((2, CH, H), jnp.float32),
            pltpu.VMEM((H,), jnp.float32),
            pltpu.VMEM((H,), jnp.float32),
            pltpu.VMEM((H,), jnp.float32),
            pltpu.SemaphoreType.DMA,
            pltpu.SemaphoreType.DMA,
            pltpu.SemaphoreType.DMA,
            pltpu.SemaphoreType.DMA,
            pltpu.SemaphoreType.DMA,
            pltpu.SemaphoreType.DMA,
            pltpu.SemaphoreType.DMA,
            pltpu.SemaphoreType.DMA,
        ],
    )
    def k(ids_hbm, wemb, ttemb, pemb, gamma, beta, out,
          ids_v, pos_v, wrows, prows, tt_v, g_v, b_v,
          sw0, sw1, sp0, sp1, so0, so1, so2, so3):
        sem_w = (sw0, sw1)
        sem_p = (sp0, sp1)
        sem_o = (so0, so1, so2, so3)
        wid = lax.axis_index("s") * NC + lax.axis_index("c")

        pltpu.sync_copy(ids_hbm.at[wid], ids_v)
        pltpu.sync_copy(ttemb.at[0], tt_v)
        pltpu.sync_copy(gamma, g_v)
        pltpu.sync_copy(beta, b_v)

        def pos_vec(j, carry):
            ids = ids_v[pl.ds(j * _L, _L)]
            m = jnp.where(ids != _PAD, jnp.int32(1), jnp.int32(0))
            cs = plsc.cumsum(m)
            pos_v[pl.ds(j * _L, _L)] = (cs + carry) * m + _PAD
            return carry + jnp.sum(m)
        lax.fori_loop(0, S // _L, pos_vec, jnp.int32(0))

        def fire_gather(g, bw, bp):
            pltpu.async_copy(wemb.at[ids_v.at[pl.ds(g * CH, CH)]],
                             wrows.at[bw], sem_w[bw % 2])
            pltpu.async_copy(pemb.at[pos_v.at[pl.ds(g * CH, CH)]],
                             prows.at[bp], sem_p[bp])

        def wait_gather(bw, bp):
            pltpu.make_async_copy(pemb.at[pl.ds(0, CH)], wrows.at[bw],
                                  sem_w[bw % 2]).wait()
            pltpu.make_async_copy(pemb.at[pl.ds(0, CH)], prows.at[bp],
                                  sem_p[bp]).wait()

        def fire_out(g, bw):
            pltpu.async_copy(wrows.at[bw], out.at[wid, pl.ds(g * CH, CH)],
                             sem_o[bw])

        def wait_out(bw):
            pltpu.make_async_copy(wrows.at[bw], out.at[wid, pl.ds(0, CH)],
                                  sem_o[bw]).wait()

        def compute(bw, bp):
            z = jnp.zeros((_L,), jnp.float32)
            for t0 in range(0, CH, T):
                def p1(h0, carry):
                    for q in range(4):
                        h = h0 * 4 + q
                        tt = tt_v[pl.ds(h * _L, _L)]
                        new = []
                        for i in range(T):
                            t = t0 + i
                            e = (wrows[bw, t, pl.ds(h * _L, _L)]
                                 + prows[bp, t, pl.ds(h * _L, _L)]
                                 + tt)
                            wrows[bw, t, pl.ds(h * _L, _L)] = e
                            new.append(carry[2 * i] + e)
                            new.append(carry[2 * i + 1] + e * e)
                        carry = tuple(new)
                    return carry

                acc = lax.fori_loop(0, NH // 4, p1, (z,) * (2 * T))

                mrs = []
                for i in range(T):
                    tot = jnp.sum(acc[2 * i])
                    tot2 = jnp.sum(acc[2 * i + 1])
                    mean = tot * (1.0 / H)
                    var = tot2 * (1.0 / H) - mean * mean
                    rstd = _rsqrt_scalar(var + _EPS)
                    mrs.append(jnp.full((_L,), mean, jnp.float32))
                    mrs.append(jnp.full((_L,), rstd, jnp.float32))

                def p2(h0, carry):
                    for q in range(4):
                        h = h0 * 4 + q
                        gv = g_v[pl.ds(h * _L, _L)]
                        bv = b_v[pl.ds(h * _L, _L)]
                        for i in range(T):
                            t = t0 + i
                            e = wrows[bw, t, pl.ds(h * _L, _L)]
                            wrows[bw, t, pl.ds(h * _L, _L)] = (
                                (e - carry[2 * i]) * carry[2 * i + 1] * gv + bv)
                    return carry

                lax.fori_loop(0, NH // 4, p2, tuple(mrs))

        fire_gather(0, 0, 0)

        def outer(go, _):
            for u in range(4):
                g = go * 4 + u

                @pl.when(g >= 3)
                def _():
                    wait_out((u + 1) % 4)

                @pl.when(g <= NCHUNK - 2)
                def _():
                    fire_gather(g + 1, (u + 1) % 4, (u + 1) % 2)

                wait_gather(u, u % 2)
                compute(u, u % 2)
                fire_out(g, u)
            return 0

        lax.fori_loop(0, NCHUNK // 4, outer, 0)
        wait_out(1)
        wait_out(2)
        wait_out(3)

    return k(input_ids, word_emb, token_type_emb, position_emb, ln_gamma, ln_beta)


# speedup vs baseline: 2.8720x; 2.8720x over previous
import functools

import jax
import jax.numpy as jnp
from jax import lax
from jax.experimental import pallas as pl
from jax.experimental.pallas import tpu as pltpu
from jax.experimental.pallas import tpu_sc as plsc

_PAD = 1
_EPS = 1e-12
_L = 16


def _rsqrt_scalar(x):
    i = lax.bitcast_convert_type(x, jnp.int32)
    y = lax.bitcast_convert_type(jnp.int32(0x5F3759DF) - (i >> 1), jnp.float32)
    for _ in range(4):
        y = y * (1.5 - 0.5 * x * y * y)
    return y


def kernel(input_ids, word_emb, token_type_emb, position_emb, ln_gamma, ln_beta):
    B, S = input_ids.shape
    V, H = word_emb.shape
    NH = H // _L

    mesh = plsc.VectorSubcoreMesh(core_axis_name="c", subcore_axis_name="s")
    NC = mesh.num_cores
    NW = NC * mesh.num_subcores
    assert B == NW, (B, NW)

    CH = 16
    NCHUNK = S // CH
    T = 8

    @functools.partial(
        pl.kernel,
        out_type=jax.ShapeDtypeStruct((B, S, H), jnp.float32),
        mesh=mesh,
        compiler_params=pltpu.CompilerParams(needs_layout_passes=False),
        scratch_types=[
            pltpu.VMEM((S,), jnp.int32),
            pltpu.VMEM((S,), jnp.int32),
            pltpu.VMEM((4, CH, H), jnp.float32),
            pltpu.VMEM((2, CH, H), jnp.float32),
            pltpu.VMEM((H,), jnp.float32),
            pltpu.VMEM((H,), jnp.float32),
            pltpu.VMEM((H,), jnp.float32),
            pltpu.SemaphoreType.DMA,
            pltpu.SemaphoreType.DMA,
            pltpu.SemaphoreType.DMA,
            pltpu.SemaphoreType.DMA,
            pltpu.SemaphoreType.DMA,
            pltpu.SemaphoreType.DMA,
            pltpu.SemaphoreType.DMA,
            pltpu.SemaphoreType.DMA,
        ],
    )
    def k(ids_hbm, wemb, ttemb, pemb, gamma, beta, out,
          ids_v, pos_v, wrows, prows, tt_v, g_v, b_v,
          sw0, sw1, sp0, sp1, so0, so1, so2, so3):
        sem_w = (sw0, sw1)
        sem_p = (sp0, sp1)
        sem_o = (so0, so1, so2, so3)
        wid = lax.axis_index("s") * NC + lax.axis_index("c")

        pltpu.sync_copy(ids_hbm.at[wid], ids_v)
        pltpu.sync_copy(ttemb.at[0], tt_v)
        pltpu.sync_copy(gamma, g_v)
        pltpu.sync_copy(beta, b_v)

        def pos_vec(j, carry):
            ids = ids_v[pl.ds(j * _L, _L)]
            m = jnp.where(ids != _PAD, jnp.int32(1), jnp.int32(0))
            cs = plsc.cumsum(m)
            pos_v[pl.ds(j * _L, _L)] = (cs + carry) * m + _PAD
            return carry + jnp.sum(m)
        lax.fori_loop(0, S // _L, pos_vec, jnp.int32(0))

        def fire_gather(g, bw, bp):
            pltpu.async_copy(wemb.at[ids_v.at[pl.ds(g * CH, CH)]],
                             wrows.at[bw], sem_w[bw % 2])
            pltpu.async_copy(pemb.at[pos_v.at[pl.ds(g * CH, CH)]],
                             prows.at[bp], sem_p[bp])

        def wait_gather(bw, bp):
            pltpu.make_async_copy(pemb.at[pl.ds(0, CH)], wrows.at[bw],
                                  sem_w[bw % 2]).wait()
            pltpu.make_async_copy(pemb.at[pl.ds(0, CH)], prows.at[bp],
                                  sem_p[bp]).wait()

        def fire_out(g, bw):
            pltpu.async_copy(wrows.at[bw], out.at[wid, pl.ds(g * CH, CH)],
                             sem_o[bw])

        def wait_out(bw):
            pltpu.make_async_copy(wrows.at[bw], out.at[wid, pl.ds(0, CH)],
                                  sem_o[bw]).wait()

        def compute(bw, bp):
            z = jnp.zeros((_L,), jnp.float32)
            for t0 in range(0, CH, T):
                @plsc.parallel_loop(0, NH, unroll=2, carry=(z,) * (2 * T))
                def acc(h, carry):
                    tt = tt_v[pl.ds(h * _L, _L)]
                    new = []
                    for i in range(T):
                        t = t0 + i
                        e = (wrows[bw, t, pl.ds(h * _L, _L)]
                             + prows[bp, t, pl.ds(h * _L, _L)]
                             + tt)
                        wrows[bw, t, pl.ds(h * _L, _L)] = e
                        new.append(carry[2 * i] + e)
                        new.append(carry[2 * i + 1] + e * e)
                    return tuple(new)

                mrs = []
                for i in range(T):
                    tot = jnp.sum(acc[2 * i])
                    tot2 = jnp.sum(acc[2 * i + 1])
                    mean = tot * (1.0 / H)
                    var = tot2 * (1.0 / H) - mean * mean
                    rstd = _rsqrt_scalar(var + _EPS)
                    mrs.append(jnp.full((_L,), mean, jnp.float32))
                    mrs.append(jnp.full((_L,), rstd, jnp.float32))

                @plsc.parallel_loop(0, NH, unroll=2, carry=tuple(mrs))
                def _p2(h, carry):
                    gv = g_v[pl.ds(h * _L, _L)]
                    bv = b_v[pl.ds(h * _L, _L)]
                    for i in range(T):
                        t = t0 + i
                        e = wrows[bw, t, pl.ds(h * _L, _L)]
                        wrows[bw, t, pl.ds(h * _L, _L)] = (
                            (e - carry[2 * i]) * carry[2 * i + 1] * gv + bv)
                    return carry

        fire_gather(0, 0, 0)

        def outer(go, _):
            for u in range(4):
                g = go * 4 + u

                @pl.when(g >= 3)
                def _():
                    wait_out((u + 1) % 4)

                @pl.when(g <= NCHUNK - 2)
                def _():
                    fire_gather(g + 1, (u + 1) % 4, (u + 1) % 2)

                wait_gather(u, u % 2)
                compute(u, u % 2)
                fire_out(g, u)
            return 0

        lax.fori_loop(0, NCHUNK // 4, outer, 0)
        wait_out(1)
        wait_out(2)
        wait_out(3)

    return k(input_ids, word_emb, token_type_emb, position_emb, ln_gamma, ln_beta)


# speedup vs baseline: 3.4032x; 1.1850x over previous
import functools

import jax
import jax.numpy as jnp
from jax import lax
from jax.experimental import pallas as pl
from jax.experimental.pallas import tpu as pltpu
from jax.experimental.pallas import tpu_sc as plsc

_PAD = 1
_EPS = 1e-12
_L = 16


def _rsqrt_scalar(x):
    i = lax.bitcast_convert_type(x, jnp.int32)
    y = lax.bitcast_convert_type(jnp.int32(0x5F3759DF) - (i >> 1), jnp.float32)
    for _ in range(4):
        y = y * (1.5 - 0.5 * x * y * y)
    return y


def kernel(input_ids, word_emb, token_type_emb, position_emb, ln_gamma, ln_beta):
    B, S = input_ids.shape
    V, H = word_emb.shape
    NH = H // _L

    mesh = plsc.VectorSubcoreMesh(core_axis_name="c", subcore_axis_name="s")
    NC = mesh.num_cores
    NW = NC * mesh.num_subcores
    assert B == NW, (B, NW)

    CH = 16
    NCHUNK = S // CH
    T = 8

    @functools.partial(
        pl.kernel,
        out_type=jax.ShapeDtypeStruct((B, S, H), jnp.float32),
        mesh=mesh,
        compiler_params=pltpu.CompilerParams(needs_layout_passes=False),
        scratch_types=[
            pltpu.VMEM((S,), jnp.int32),
            pltpu.VMEM((S,), jnp.int32),
            pltpu.VMEM((4, CH, H), jnp.float32),
            pltpu.VMEM((2, CH, H), jnp.float32),
            pltpu.VMEM((H,), jnp.float32),
            pltpu.VMEM((H,), jnp.float32),
            pltpu.VMEM((H,), jnp.float32),
            pltpu.SemaphoreType.DMA,
            pltpu.SemaphoreType.DMA,
            pltpu.SemaphoreType.DMA,
            pltpu.SemaphoreType.DMA,
            pltpu.SemaphoreType.DMA,
            pltpu.SemaphoreType.DMA,
            pltpu.SemaphoreType.DMA,
            pltpu.SemaphoreType.DMA,
        ],
    )
    def k(ids_hbm, wemb, ttemb, pemb, gamma, beta, out,
          ids_v, pos_v, wrows, prows, tt_v, g_v, b_v,
          sw0, sw1, sp0, sp1, so0, so1, so2, so3):
        sem_w = (sw0, sw1)
        sem_p = (sp0, sp1)
        sem_o = (so0, so1, so2, so3)
        wid = lax.axis_index("s") * NC + lax.axis_index("c")

        pltpu.sync_copy(ids_hbm.at[wid], ids_v)
        pltpu.sync_copy(ttemb.at[0], tt_v)
        pltpu.sync_copy(gamma, g_v)
        pltpu.sync_copy(beta, b_v)

        def pos_vec(j, carry):
            ids = ids_v[pl.ds(j * _L, _L)]
            m = jnp.where(ids != _PAD, jnp.int32(1), jnp.int32(0))
            cs = plsc.cumsum(m)
            pos_v[pl.ds(j * _L, _L)] = (cs + carry) * m + _PAD
            return carry + jnp.sum(m)
        lax.fori_loop(0, S // _L, pos_vec, jnp.int32(0))

        def fire_gather(g, bw, bp):
            pltpu.async_copy(wemb.at[ids_v.at[pl.ds(g * CH, CH)]],
                             wrows.at[bw], sem_w[bw % 2])
            pltpu.async_copy(pemb.at[pos_v.at[pl.ds(g * CH, CH)]],
                             prows.at[bp], sem_p[bp])

        def wait_gather(bw, bp):
            pltpu.make_async_copy(pemb.at[pl.ds(0, CH)], wrows.at[bw],
                                  sem_w[bw % 2]).wait()
            pltpu.make_async_copy(pemb.at[pl.ds(0, CH)], prows.at[bp],
                                  sem_p[bp]).wait()

        def fire_out(g, bw):
            pltpu.async_copy(wrows.at[bw], out.at[wid, pl.ds(g * CH, CH)],
                             sem_o[bw])

        def wait_out(bw):
            pltpu.make_async_copy(wrows.at[bw], out.at[wid, pl.ds(0, CH)],
                                  sem_o[bw]).wait()

        def compute(bw, bp):
            z = jnp.zeros((_L,), jnp.float32)
            for t0 in range(0, CH, T):
                @plsc.parallel_loop(0, NH, carry=(z,) * (2 * T))
                def acc(h, carry):
                    tt = tt_v[pl.ds(h * _L, _L)]
                    new = []
                    for i in range(T):
                        t = t0 + i
                        e = (wrows[bw, t, pl.ds(h * _L, _L)]
                             + prows[bp, t, pl.ds(h * _L, _L)]
                             + tt)
                        wrows[bw, t, pl.ds(h * _L, _L)] = e
                        new.append(carry[2 * i] + e)
                        new.append(carry[2 * i + 1] + e * e)
                    return tuple(new)

                mrs = []
                for i in range(T):
                    tot = jnp.sum(acc[2 * i])
                    tot2 = jnp.sum(acc[2 * i + 1])
                    mean = tot * (1.0 / H)
                    var = tot2 * (1.0 / H) - mean * mean
                    rstd = _rsqrt_scalar(var + _EPS)
                    mrs.append(jnp.full((_L,), mean, jnp.float32))
                    mrs.append(jnp.full((_L,), rstd, jnp.float32))

                @plsc.parallel_loop(0, NH, carry=tuple(mrs))
                def _p2(h, carry):
                    gv = g_v[pl.ds(h * _L, _L)]
                    bv = b_v[pl.ds(h * _L, _L)]
                    for i in range(T):
                        t = t0 + i
                        e = wrows[bw, t, pl.ds(h * _L, _L)]
                        wrows[bw, t, pl.ds(h * _L, _L)] = (
                            (e - carry[2 * i]) * carry[2 * i + 1] * gv + bv)
                    return carry

        fire_gather(0, 0, 0)

        def outer(go, _):
            for u in range(4):
                g = go * 4 + u

                @pl.when(g >= 3)
                def _():
                    wait_out((u + 1) % 4)

                @pl.when(g <= NCHUNK - 2)
                def _():
                    fire_gather(g + 1, (u + 1) % 4, (u + 1) % 2)

                wait_gather(u, u % 2)
                compute(u, u % 2)
                fire_out(g, u)
            return 0

        lax.fori_loop(0, NCHUNK // 4, outer, 0)
        wait_out(1)
        wait_out(2)
        wait_out(3)

    return k(input_ids, word_emb, token_type_emb, position_emb, ln_gamma, ln_beta)


# speedup vs baseline: 3.4590x; 1.0164x over previous
import functools

import jax
import jax.numpy as jnp
from jax import lax
from jax.experimental import pallas as pl
from jax.experimental.pallas import tpu as pltpu
from jax.experimental.pallas import tpu_sc as plsc

_PAD = 1
_EPS = 1e-12
_L = 16


def _rsqrt_scalar(x):
    i = lax.bitcast_convert_type(x, jnp.int32)
    y = lax.bitcast_convert_type(jnp.int32(0x5F3759DF) - (i >> 1), jnp.float32)
    for _ in range(4):
        y = y * (1.5 - 0.5 * x * y * y)
    return y


def kernel(input_ids, word_emb, token_type_emb, position_emb, ln_gamma, ln_beta):
    B, S = input_ids.shape
    V, H = word_emb.shape
    NH = H // _L

    mesh = plsc.VectorSubcoreMesh(core_axis_name="c", subcore_axis_name="s")
    NC = mesh.num_cores
    NW = NC * mesh.num_subcores
    assert B == NW, (B, NW)

    CH = 16
    NCHUNK = S // CH
    T = 16

    @functools.partial(
        pl.kernel,
        out_type=jax.ShapeDtypeStruct((B, S, H), jnp.float32),
        mesh=mesh,
        compiler_params=pltpu.CompilerParams(needs_layout_passes=False),
        scratch_types=[
            pltpu.VMEM((S,), jnp.int32),
            pltpu.VMEM((S,), jnp.int32),
            pltpu.VMEM((4, CH, H), jnp.float32),
            pltpu.VMEM((2, CH, H), jnp.float32),
            pltpu.VMEM((H,), jnp.float32),
            pltpu.VMEM((H,), jnp.float32),
            pltpu.VMEM((H,), jnp.float32),
            pltpu.SemaphoreType.DMA,
            pltpu.SemaphoreType.DMA,
            pltpu.SemaphoreType.DMA,
            pltpu.SemaphoreType.DMA,
            pltpu.SemaphoreType.DMA,
            pltpu.SemaphoreType.DMA,
            pltpu.SemaphoreType.DMA,
            pltpu.SemaphoreType.DMA,
        ],
    )
    def k(ids_hbm, wemb, ttemb, pemb, gamma, beta, out,
          ids_v, pos_v, wrows, prows, tt_v, g_v, b_v,
          sw0, sw1, sp0, sp1, so0, so1, so2, so3):
        sem_w = (sw0, sw1)
        sem_p = (sp0, sp1)
        sem_o = (so0, so1, so2, so3)
        wid = lax.axis_index("s") * NC + lax.axis_index("c")

        pltpu.sync_copy(ids_hbm.at[wid], ids_v)
        pltpu.sync_copy(ttemb.at[0], tt_v)
        pltpu.sync_copy(gamma, g_v)
        pltpu.sync_copy(beta, b_v)

        def pos_vec(j, carry):
            ids = ids_v[pl.ds(j * _L, _L)]
            m = jnp.where(ids != _PAD, jnp.int32(1), jnp.int32(0))
            cs = plsc.cumsum(m)
            pos_v[pl.ds(j * _L, _L)] = (cs + carry) * m + _PAD
            return carry + jnp.sum(m)
        lax.fori_loop(0, S // _L, pos_vec, jnp.int32(0))

        def fire_gather(g, bw, bp):
            pltpu.async_copy(wemb.at[ids_v.at[pl.ds(g * CH, CH)]],
                             wrows.at[bw], sem_w[bw % 2])
            pltpu.async_copy(pemb.at[pos_v.at[pl.ds(g * CH, CH)]],
                             prows.at[bp], sem_p[bp])

        def wait_gather(bw, bp):
            pltpu.make_async_copy(pemb.at[pl.ds(0, CH)], wrows.at[bw],
                                  sem_w[bw % 2]).wait()
            pltpu.make_async_copy(pemb.at[pl.ds(0, CH)], prows.at[bp],
                                  sem_p[bp]).wait()

        def fire_out(g, bw):
            pltpu.async_copy(wrows.at[bw], out.at[wid, pl.ds(g * CH, CH)],
                             sem_o[bw])

        def wait_out(bw):
            pltpu.make_async_copy(wrows.at[bw], out.at[wid, pl.ds(0, CH)],
                                  sem_o[bw]).wait()

        def compute(bw, bp):
            z = jnp.zeros((_L,), jnp.float32)
            for t0 in range(0, CH, T):
                @plsc.parallel_loop(0, NH, carry=(z,) * (2 * T))
                def acc(h, carry):
                    tt = tt_v[pl.ds(h * _L, _L)]
                    new = []
                    for i in range(T):
                        t = t0 + i
                        e = (wrows[bw, t, pl.ds(h * _L, _L)]
                             + prows[bp, t, pl.ds(h * _L, _L)]
                             + tt)
                        wrows[bw, t, pl.ds(h * _L, _L)] = e
                        new.append(carry[2 * i] + e)
                        new.append(carry[2 * i + 1] + e * e)
                    return tuple(new)

                mrs = []
                for i in range(T):
                    tot = jnp.sum(acc[2 * i])
                    tot2 = jnp.sum(acc[2 * i + 1])
                    mean = tot * (1.0 / H)
                    var = tot2 * (1.0 / H) - mean * mean
                    rstd = _rsqrt_scalar(var + _EPS)
                    mrs.append(jnp.full((_L,), mean, jnp.float32))
                    mrs.append(jnp.full((_L,), rstd, jnp.float32))

                @plsc.parallel_loop(0, NH, carry=tuple(mrs))
                def _p2(h, carry):
                    gv = g_v[pl.ds(h * _L, _L)]
                    bv = b_v[pl.ds(h * _L, _L)]
                    for i in range(T):
                        t = t0 + i
                        e = wrows[bw, t, pl.ds(h * _L, _L)]
                        wrows[bw, t, pl.ds(h * _L, _L)] = (
                            (e - carry[2 * i]) * carry[2 * i + 1] * gv + bv)
                    return carry

        fire_gather(0, 0, 0)

        def outer(go, _):
            for u in range(4):
                g = go * 4 + u

                @pl.when(g >= 3)
                def _():
                    wait_out((u + 1) % 4)

                @pl.when(g <= NCHUNK - 2)
                def _():
                    fire_gather(g + 1, (u + 1) % 4, (u + 1) % 2)

                wait_gather(u, u % 2)
                compute(u, u % 2)
                fire_out(g, u)
            return 0

        lax.fori_loop(0, NCHUNK // 4, outer, 0)
        wait_out(1)
        wait_out(2)
        wait_out(3)

    return k(input_ids, word_emb, token_type_emb, position_emb, ln_gamma, ln_beta)
